# baseline (speedup 1.0000x reference)
import functools

import jax
import jax.numpy as jnp
from jax import lax
from jax.experimental import pallas as pl
from jax.experimental.pallas import tpu as pltpu
from jax.experimental.pallas import tpu_sc as plsc

NC = 2
NS = 16
NW = NC * NS
L = 16

CHUNK = 128
NBUF = 4
LEAD = 3


def _make_gather(b_total, s_total, d, n_chunks):
  mesh = plsc.VectorSubcoreMesh(core_axis_name="c", subcore_axis_name="s")
  assert n_chunks % NBUF == 0
  n_outer = n_chunks // NBUF
  bt_per_s = b_total // CHUNK
  assert bt_per_s & (bt_per_s - 1) == 0
  bt_bits = bt_per_s.bit_length() - 1
  ngrp = CHUNK // L

  @functools.partial(
      pl.kernel,
      out_type=jax.ShapeDtypeStruct((s_total, d, b_total), jnp.float32),
      mesh=mesh,
      compiler_params=pltpu.CompilerParams(needs_layout_passes=False),
      scratch_types=[
          pltpu.VMEM((NBUF, CHUNK), jnp.int32),
          pltpu.VMEM((NBUF, CHUNK), jnp.int32),
          pltpu.VMEM((NBUF, CHUNK, 128), jnp.float32),
          pltpu.VMEM((NBUF, d, CHUNK), jnp.float32),
          [pltpu.SemaphoreType.DMA] * NBUF,
          [pltpu.SemaphoreType.DMA] * NBUF,
          [pltpu.SemaphoreType.DMA] * NBUF,
      ],
  )
  def gather(table_hbm, tok_hbm, out_hbm, idx_r, rows_v, g_v, w_v,
             isems, gsems, wsems):
    wid = lax.axis_index("s") * NC + lax.axis_index("c")
    j0w = wid * n_chunks
    lane = jnp.arange(L, dtype=jnp.int32)

    def chunk_pos(c):
      j = j0w + c
      return j >> bt_bits, (j & (bt_per_s - 1)) * CHUNK

    def tok_slice(c):
      s, b0 = chunk_pos(c)
      return tok_hbm.at[s, pl.ds(b0, CHUNK)]

    def start_idx_load(c, slot):
      pltpu.async_copy(tok_slice(c), idx_r.at[slot], isems[slot])

    def wait_idx_load(slot):
      pltpu.make_async_copy(tok_slice(0), idx_r.at[slot], isems[slot]).wait()

    def compute_rows(slot):
      for grp in range(ngrp):
        rows_v[slot, pl.ds(grp * L, L)] = idx_r[slot, pl.ds(grp * L, L)]

    def start_gather(slot):
      pltpu.async_copy(table_hbm.at[rows_v.at[slot]], g_v.at[slot],
                       gsems[slot])

    def wait_gather(slot):
      pltpu.make_async_copy(table_hbm.at[rows_v.at[0]], g_v.at[slot],
                            gsems[slot]).wait()

    def out_slice(c):
      s, b0 = chunk_pos(c)
      return out_hbm.at[s, :, pl.ds(b0, CHUNK)]

    def start_writeback(c, slot):
      pltpu.async_copy(w_v.at[slot], out_slice(c), wsems[slot])

    def wait_writeback(slot):
      pltpu.make_async_copy(w_v.at[slot], out_slice(0), wsems[slot]).wait()

    def transpose_chunk(slot):
      g = g_v.at[slot]
      rows = [lane + (grp * L) for grp in range(ngrp)]
      zero = lane & 0

      @plsc.parallel_loop(0, d, unroll=16)
      def _(dd):
        cold = zero + dd
        for grp in range(ngrp):
          v = plsc.load_gather(g, [rows[grp], cold])
          w_v[slot, dd, pl.ds(grp * L, L)] = v

    for k in range(NBUF):
      start_idx_load(k, k)
    for k in range(LEAD):
      wait_idx_load(k)
      compute_rows(k)
      start_gather(k)
      start_idx_load(k + NBUF, k)

    def outer(o, carry):
      c0 = o * NBUF
      for i in range(NBUF):
        c = c0 + i
        nslot = (i + LEAD) % NBUF

        @pl.when(c + LEAD < n_chunks)
        def _():
          wait_idx_load(nslot)
          compute_rows(nslot)
          start_gather(nslot)
          @pl.when(c + LEAD + NBUF < n_chunks)
          def _():
            start_idx_load(c + LEAD + NBUF, nslot)

        wait_gather(i)
        @pl.when(o >= 1)
        def _():
          wait_writeback(i)
        transpose_chunk(i)
        start_writeback(c, i)
      return carry

    lax.fori_loop(0, n_outer, outer, 0)

    for i in range(NBUF):
      wait_writeback(i)

  return gather


def kernel(tokens, table):
  b, s = tokens.shape
  vocab, d = table.shape
  total = b * s
  assert total % (NW * CHUNK) == 0 and d == 64
  n_chunks = total // (NW * CHUNK)
  table2 = jnp.pad(table, ((0, 0), (0, d)))
  outp = _make_gather(b, s, d, n_chunks)(table2, tokens.T.astype(jnp.int32))
  return outp.transpose(2, 0, 1)


# speedup vs baseline: 1.0030x; 1.0030x over previous
import functools

import jax
import jax.numpy as jnp
from jax import lax
from jax.experimental import pallas as pl
from jax.experimental.pallas import tpu as pltpu
from jax.experimental.pallas import tpu_sc as plsc

NC = 2
NS = 16
NW = NC * NS
L = 16

CHUNK = 128
NBUF = 4
LEAD = 3


def _make_gather(b_total, s_total, d, n_chunks):
  mesh = plsc.VectorSubcoreMesh(core_axis_name="c", subcore_axis_name="s")
  assert n_chunks % NBUF == 0
  n_outer = n_chunks // NBUF
  bt_per_s = b_total // CHUNK
  assert bt_per_s & (bt_per_s - 1) == 0
  bt_bits = bt_per_s.bit_length() - 1
  ngrp = CHUNK // L

  @functools.partial(
      pl.kernel,
      out_type=jax.ShapeDtypeStruct((s_total, d, b_total), jnp.float32),
      mesh=mesh,
      compiler_params=pltpu.CompilerParams(needs_layout_passes=False),
      scratch_types=[
          pltpu.VMEM((NBUF, CHUNK), jnp.int32),
          pltpu.VMEM((NBUF, CHUNK), jnp.int32),
          pltpu.VMEM((NBUF, CHUNK, 128), jnp.float32),
          pltpu.VMEM((NBUF, d, CHUNK), jnp.float32),
          [pltpu.SemaphoreType.DMA] * NBUF,
          [pltpu.SemaphoreType.DMA] * NBUF,
          [pltpu.SemaphoreType.DMA] * NBUF,
      ],
  )
  def gather(table_hbm, tok_hbm, out_hbm, idx_r, rows_v, g_v, w_v,
             isems, gsems, wsems):
    wid = lax.axis_index("s") * NC + lax.axis_index("c")
    j0w = wid * n_chunks
    lane = jnp.arange(L, dtype=jnp.int32)

    def chunk_pos(c):
      j = j0w + c
      return j >> bt_bits, (j & (bt_per_s - 1)) * CHUNK

    def tok_slice(c):
      s, b0 = chunk_pos(c)
      return tok_hbm.at[s, pl.ds(b0, CHUNK)]

    def start_idx_load(c, slot):
      pltpu.async_copy(tok_slice(c), idx_r.at[slot], isems[slot])

    def wait_idx_load(slot):
      pltpu.make_async_copy(tok_slice(0), idx_r.at[slot], isems[slot]).wait()

    def compute_rows(slot):
      for grp in range(ngrp):
        rows_v[slot, pl.ds(grp * L, L)] = idx_r[slot, pl.ds(grp * L, L)]

    def start_gather(slot):
      pltpu.async_copy(table_hbm.at[rows_v.at[slot]], g_v.at[slot],
                       gsems[slot])

    def wait_gather(slot):
      pltpu.make_async_copy(table_hbm.at[rows_v.at[0]], g_v.at[slot],
                            gsems[slot]).wait()

    def out_slice(c):
      s, b0 = chunk_pos(c)
      return out_hbm.at[s, :, pl.ds(b0, CHUNK)]

    def start_writeback(c, slot):
      pltpu.async_copy(w_v.at[slot], out_slice(c), wsems[slot])

    def wait_writeback(slot):
      pltpu.make_async_copy(w_v.at[slot], out_slice(0), wsems[slot]).wait()

    def transpose_chunk(slot):
      g = g_v.at[slot]
      rows = [lane + (grp * L) for grp in range(ngrp)]
      zero = lane & 0

      @plsc.parallel_loop(0, d, unroll=8)
      def _(dd):
        cold = zero + dd
        for grp in range(ngrp):
          v = plsc.load_gather(g, [rows[grp], cold])
          w_v[slot, dd, pl.ds(grp * L, L)] = v

    for k in range(NBUF):
      start_idx_load(k, k)
    for k in range(LEAD):
      wait_idx_load(k)
      compute_rows(k)
      start_gather(k)
      start_idx_load(k + NBUF, k)

    def outer(o, carry):
      c0 = o * NBUF
      for i in range(NBUF):
        c = c0 + i
        nslot = (i + LEAD) % NBUF

        @pl.when(c + LEAD < n_chunks)
        def _():
          wait_idx_load(nslot)
          compute_rows(nslot)
          start_gather(nslot)
          @pl.when(c + LEAD + NBUF < n_chunks)
          def _():
            start_idx_load(c + LEAD + NBUF, nslot)

        wait_gather(i)
        @pl.when(o >= 1)
        def _():
          wait_writeback(i)
        transpose_chunk(i)
        start_writeback(c, i)
      return carry

    lax.fori_loop(0, n_outer, outer, 0)

    for i in range(NBUF):
      wait_writeback(i)

  return gather


def kernel(tokens, table):
  b, s = tokens.shape
  vocab, d = table.shape
  total = b * s
  assert total % (NW * CHUNK) == 0 and d == 64
  n_chunks = total // (NW * CHUNK)
  table2 = jnp.pad(table, ((0, 0), (0, d)))
  outp = _make_gather(b, s, d, n_chunks)(table2, tokens.T.astype(jnp.int32))
  return outp.transpose(2, 0, 1)
